# Initial kernel scaffold; baseline (speedup 1.0000x reference)
import jax
import jax.numpy as jnp
from jax.experimental import pallas as pl


def kernel(p, x, o, Wq, bq, Wk, bk, Wv, bv, Wp1, bp1, gp, bp, Wp2, bp2, gw1, bw1, Ww1, bww1, gw2, bw2, Ww2, bww2):
    raise NotImplementedError("write your pallas kernel here")


# speedup vs baseline: 3.4246x; 3.4246x over previous
import functools

import jax
import jax.numpy as jnp
from jax import lax
from jax.experimental import pallas as pl
from jax.experimental.pallas import tpu as pltpu
from jax.experimental.pallas import tpu_sc as plsc

NREAL = 10000
NPAD = 10240
C = 256
NS = 16
SP = 8
CS = 32
BLK = 128
NBLK = NPAD // BLK
FAR = 1.0e6
BIGI = 1 << 30
EPS = 1e-5


def _qkv_body(x_ref, wq_ref, wk_ref, wv_ref, b3_ref, q_ref, k_ref, v_ref):
    xb = x_ref[...]
    q_ref[...] = jnp.dot(xb, wq_ref[...], preferred_element_type=jnp.float32) + b3_ref[0:1, :]
    k_ref[...] = jnp.dot(xb, wk_ref[...], preferred_element_type=jnp.float32) + b3_ref[1:2, :]
    v_ref[...] = jnp.dot(xb, wv_ref[...], preferred_element_type=jnp.float32) + b3_ref[2:3, :]


def _run_qkv(x_pad, Wq, Wk, Wv, b3):
    out = jax.ShapeDtypeStruct((NPAD, C), jnp.float32)
    return pl.pallas_call(
        _qkv_body,
        grid=(NBLK,),
        in_specs=[
            pl.BlockSpec((BLK, C), lambda i: (i, 0)),
            pl.BlockSpec((C, C), lambda i: (0, 0)),
            pl.BlockSpec((C, C), lambda i: (0, 0)),
            pl.BlockSpec((C, C), lambda i: (0, 0)),
            pl.BlockSpec((8, C), lambda i: (0, 0)),
        ],
        out_specs=[
            pl.BlockSpec((BLK, C), lambda i: (i, 0)),
            pl.BlockSpec((BLK, C), lambda i: (i, 0)),
            pl.BlockSpec((BLK, C), lambda i: (i, 0)),
        ],
        out_shape=[out, out, out],
    )(x_pad, Wq, Wk, Wv, b3)


def _knn_body(pb_ref, pt_ref, pf_ref, idx_ref, prel_ref):
    pb = pb_ref[...]
    pt = pt_ref[...]
    sqb = jnp.sum(pb * pb, axis=1, keepdims=True)
    sqa = jnp.sum(pt * pt, axis=0, keepdims=True)
    d2 = sqb + sqa - 2.0 * jnp.dot(pb, pt, preferred_element_type=jnp.float32)
    li = lax.broadcasted_iota(jnp.int32, (BLK, NPAD), 1)
    cur = d2
    for t in range(NS):
        m = jnp.min(cur, axis=1, keepdims=True)
        cand = jnp.where(cur <= m, li, BIGI)
        am = jnp.min(cand, axis=1, keepdims=True)
        idx_ref[:, t:t + 1] = am
        hit = li == am
        oh = hit.astype(jnp.float32)
        pam = jnp.dot(oh, pf_ref[...], preferred_element_type=jnp.float32)
        prel_ref[:, t:t + 1, :] = (pam - pb)[:, None, :]
        cur = jnp.where(hit, jnp.float32(3e38), cur)


def _run_knn(p16, p16t):
    return pl.pallas_call(
        _knn_body,
        grid=(NBLK,),
        in_specs=[
            pl.BlockSpec((BLK, 16), lambda i: (i, 0)),
            pl.BlockSpec((16, NPAD), lambda i: (0, 0)),
            pl.BlockSpec((NPAD, 16), lambda i: (0, 0)),
        ],
        out_specs=[
            pl.BlockSpec((BLK, NS), lambda i: (i, 0)),
            pl.BlockSpec((BLK, NS, 16), lambda i: (i, 0, 0)),
        ],
        out_shape=[
            jax.ShapeDtypeStruct((NPAD, NS), jnp.int32),
            jax.ShapeDtypeStruct((NPAD, NS, 16), jnp.float32),
        ],
    )(p16, p16t, p16)


def _sc_gather(xk, xv, idx_flat):
    info = plsc.get_sparse_core_info()
    nw = info.num_cores * info.num_subcores
    B = idx_flat.shape[0]
    b_per_w = B // nw
    CH = 128
    n_ch = b_per_w // CH
    mesh = plsc.VectorSubcoreMesh(core_axis_name="c", subcore_axis_name="s")

    @functools.partial(
        pl.kernel, mesh=mesh,
        out_type=[
            jax.ShapeDtypeStruct((B, C), jnp.float32),
            jax.ShapeDtypeStruct((B, C), jnp.float32),
        ],
        scratch_types=[
            pltpu.VMEM((CH,), jnp.int32),
            pltpu.VMEM((CH, C), jnp.float32),
            pltpu.VMEM((CH, C), jnp.float32),
            pltpu.SemaphoreType.DMA,
            pltpu.SemaphoreType.DMA,
        ],
    )
    def k(tk_hbm, tv_hbm, idx_hbm, ok_hbm, ov_hbm,
          idx_v, rk_v, rv_v, sk, sv):
        wid = lax.axis_index("s") * info.num_cores + lax.axis_index("c")
        base = wid * b_per_w

        def body(i, carry):
            off = base + i * CH
            pltpu.sync_copy(idx_hbm.at[pl.ds(off, CH)], idx_v)
            ck = pltpu.async_copy(tk_hbm.at[idx_v], rk_v, sk)
            cv = pltpu.async_copy(tv_hbm.at[idx_v], rv_v, sv)
            ck.wait()
            cv.wait()
            pltpu.sync_copy(rk_v, ok_hbm.at[pl.ds(off, CH)])
            pltpu.sync_copy(rv_v, ov_hbm.at[pl.ds(off, CH)])
            return carry

        lax.fori_loop(0, n_ch, body, 0)

    return k(xk, xv, idx_flat)


def _pr_block(prel_ref, wp1_ref, wp2_ref, bp_ref):
    prel2 = prel_ref[...].reshape(BLK * NS, 16)
    y = jnp.dot(prel2, wp1_ref[...], preferred_element_type=jnp.float32) + bp_ref[0:1, 0:16]
    y = jnp.maximum(y, 0.0)
    return jnp.dot(y, wp2_ref[...], preferred_element_type=jnp.float32) + bp_ref[1:2, :]


def _rowmask(blk_i):
    pid = lax.broadcasted_iota(jnp.int32, (BLK * NS, 1), 0) // NS + blk_i * BLK
    return jnp.where(pid < NREAL, 1.0, 0.0).astype(jnp.float32)


def _p0_body(prel_ref, wp1_ref, bp1_ref, s_ref):
    @pl.when(pl.program_id(0) == 0)
    def _():
        s_ref[...] = jnp.zeros_like(s_ref)

    prel2 = prel_ref[...].reshape(BLK * NS, 16)
    y = jnp.dot(prel2, wp1_ref[...], preferred_element_type=jnp.float32) + bp1_ref[0:1, :]
    mask = _rowmask(pl.program_id(0))
    ym = y * mask
    s_ref[0:1, :] += jnp.sum(ym, axis=0, keepdims=True)
    s_ref[1:2, :] += jnp.sum(ym * y, axis=0, keepdims=True)


def _run_p0(prel3, Wp1_16, bp1_16):
    return pl.pallas_call(
        _p0_body,
        grid=(NBLK,),
        in_specs=[
            pl.BlockSpec((BLK, NS, 16), lambda i: (i, 0, 0)),
            pl.BlockSpec((16, 16), lambda i: (0, 0)),
            pl.BlockSpec((8, 16), lambda i: (0, 0)),
        ],
        out_specs=pl.BlockSpec((8, 16), lambda i: (0, 0)),
        out_shape=jax.ShapeDtypeStruct((8, 16), jnp.float32),
    )(prel3, Wp1_16, bp1_16)


def _p1_body(xkg_ref, xq_ref, prel_ref, wp1_ref, wp2_ref, bp_ref, s_ref):
    @pl.when(pl.program_id(0) == 0)
    def _():
        s_ref[...] = jnp.zeros_like(s_ref)

    pr = _pr_block(prel_ref, wp1_ref, wp2_ref, bp_ref)
    xkg2 = xkg_ref[...].reshape(BLK * NS, C)
    xq2 = jnp.broadcast_to(xq_ref[...][:, None, :], (BLK, NS, C)).reshape(BLK * NS, C)
    r = xkg2 - xq2 + pr
    mask = _rowmask(pl.program_id(0))
    rm = r * mask
    s_ref[0:1, :] += jnp.sum(rm, axis=0, keepdims=True)
    s_ref[1:2, :] += jnp.sum(rm * r, axis=0, keepdims=True)


def _run_p1(xkg3, xq, prel3, Wp1f, Wp2_16, bpf):
    return pl.pallas_call(
        _p1_body,
        grid=(NBLK,),
        in_specs=[
            pl.BlockSpec((BLK, NS, C), lambda i: (i, 0, 0)),
            pl.BlockSpec((BLK, C), lambda i: (i, 0)),
            pl.BlockSpec((BLK, NS, 16), lambda i: (i, 0, 0)),
            pl.BlockSpec((16, 16), lambda i: (0, 0)),
            pl.BlockSpec((16, C), lambda i: (0, 0)),
            pl.BlockSpec((8, C), lambda i: (0, 0)),
        ],
        out_specs=pl.BlockSpec((8, C), lambda i: (0, 0)),
        out_shape=jax.ShapeDtypeStruct((8, C), jnp.float32),
    )(xkg3, xq, prel3, Wp1f, Wp2_16, bpf)


def _p2_body(xkg_ref, xq_ref, prel_ref, wp1_ref, wp2_ref, bp_ref,
             st2_ref, ww1_ref, u_ref, s_ref):
    @pl.when(pl.program_id(0) == 0)
    def _():
        s_ref[...] = jnp.zeros_like(s_ref)

    pr = _pr_block(prel_ref, wp1_ref, wp2_ref, bp_ref)
    xkg2 = xkg_ref[...].reshape(BLK * NS, C)
    xq2 = jnp.broadcast_to(xq_ref[...][:, None, :], (BLK, NS, C)).reshape(BLK * NS, C)
    r = xkg2 - xq2 + pr
    rn = jnp.maximum(r * st2_ref[0:1, :] + st2_ref[1:2, :], 0.0)
    u = jnp.dot(rn, ww1_ref[...], preferred_element_type=jnp.float32) + st2_ref[2:3, 0:CS]
    u_ref[...] = u.reshape(BLK, NS, CS)
    mask = _rowmask(pl.program_id(0))
    um = u * mask
    s_ref[0:1, :] += jnp.sum(um, axis=0, keepdims=True)
    s_ref[1:2, :] += jnp.sum(um * u, axis=0, keepdims=True)


def _run_p2(xkg3, xq, prel3, Wp1f, Wp2_16, bpf, st2, Ww1):
    return pl.pallas_call(
        _p2_body,
        grid=(NBLK,),
        in_specs=[
            pl.BlockSpec((BLK, NS, C), lambda i: (i, 0, 0)),
            pl.BlockSpec((BLK, C), lambda i: (i, 0)),
            pl.BlockSpec((BLK, NS, 16), lambda i: (i, 0, 0)),
            pl.BlockSpec((16, 16), lambda i: (0, 0)),
            pl.BlockSpec((16, C), lambda i: (0, 0)),
            pl.BlockSpec((8, C), lambda i: (0, 0)),
            pl.BlockSpec((8, C), lambda i: (0, 0)),
            pl.BlockSpec((C, CS), lambda i: (0, 0)),
        ],
        out_specs=[
            pl.BlockSpec((BLK, NS, CS), lambda i: (i, 0, 0)),
            pl.BlockSpec((8, CS), lambda i: (0, 0)),
        ],
        out_shape=[
            jax.ShapeDtypeStruct((NPAD, NS, CS), jnp.float32),
            jax.ShapeDtypeStruct((8, CS), jnp.float32),
        ],
    )(xkg3, xq, prel3, Wp1f, Wp2_16, bpf, st2, Ww1)


def _p3_body(u_ref, xvg_ref, prel_ref, wp1_ref, wp2_ref, bp_ref,
             st3_ref, ww2_ref, o_ref):
    u = u_ref[...].reshape(BLK * NS, CS)
    un = jnp.maximum(u * st3_ref[0:1, 0:CS] + st3_ref[1:2, 0:CS], 0.0)
    w = jnp.dot(un, ww2_ref[...], preferred_element_type=jnp.float32) + st3_ref[2:3, 0:CS]
    w3 = w.reshape(BLK, NS, CS)
    wmax = jnp.max(w3, axis=1, keepdims=True)
    we = jnp.exp(w3 - wmax)
    wsm = we / jnp.sum(we, axis=1, keepdims=True)

    pr = _pr_block(prel_ref, wp1_ref, wp2_ref, bp_ref)
    val = xvg_ref[...] + pr.reshape(BLK, NS, C)
    for s in range(SP):
        seg = val[:, :, s * CS:(s + 1) * CS] * wsm
        o_ref[:, s * CS:(s + 1) * CS] = jnp.sum(seg, axis=1)


def _run_p3(u3, xvg3, prel3, Wp1f, Wp2_16, bpf, st3, Ww2):
    return pl.pallas_call(
        _p3_body,
        grid=(NBLK,),
        in_specs=[
            pl.BlockSpec((BLK, NS, CS), lambda i: (i, 0, 0)),
            pl.BlockSpec((BLK, NS, C), lambda i: (i, 0, 0)),
            pl.BlockSpec((BLK, NS, 16), lambda i: (i, 0, 0)),
            pl.BlockSpec((16, 16), lambda i: (0, 0)),
            pl.BlockSpec((16, C), lambda i: (0, 0)),
            pl.BlockSpec((8, C), lambda i: (0, 0)),
            pl.BlockSpec((8, CS), lambda i: (0, 0)),
            pl.BlockSpec((CS, CS), lambda i: (0, 0)),
        ],
        out_specs=pl.BlockSpec((BLK, C), lambda i: (i, 0)),
        out_shape=jax.ShapeDtypeStruct((NPAD, C), jnp.float32),
    )(u3, xvg3, prel3, Wp1f, Wp2_16, bpf, st3, Ww2)


def kernel(p, x, o, Wq, bq, Wk, bk, Wv, bv, Wp1, bp1, gp, bp, Wp2, bp2,
           gw1, bw1, Ww1, bww1, gw2, bw2, Ww2, bww2):
    f32 = jnp.float32
    npad = NPAD - NREAL

    x_pad = jnp.pad(x, ((0, npad), (0, 0)))
    p16 = jnp.pad(p, ((0, npad), (0, 13)), constant_values=0.0)
    p16 = p16.at[NREAL:, 0:3].set(FAR)
    p16t = p16.T

    b3 = jnp.zeros((8, C), f32).at[0].set(bq).at[1].set(bk).at[2].set(bv)

    Wp1_16 = jnp.zeros((16, 16), f32).at[0:3, 0:3].set(Wp1)
    bp1_16 = jnp.zeros((8, 16), f32).at[0, 0:3].set(bp1)
    Wp2_16 = jnp.zeros((16, C), f32).at[0:3, :].set(Wp2)

    xq, xk, xv = _run_qkv(x_pad, Wq, Wk, Wv, b3)

    idx, prel3 = _run_knn(p16, p16t)

    xkg, xvg = _sc_gather(xk, xv, idx.reshape(-1))
    xkg3 = xkg.reshape(NPAD, NS, C)
    xvg3 = xvg.reshape(NPAD, NS, C)

    s1 = _run_p0(prel3, Wp1_16, bp1_16)
    cnt = jnp.float32(NREAL * NS)
    m1 = s1[0] / cnt
    v1 = s1[1] / cnt - m1 * m1
    gp16 = jnp.ones((16,), f32).at[0:3].set(gp)
    bpv16 = jnp.zeros((16,), f32).at[0:3].set(bp)
    sc1 = gp16 / jnp.sqrt(v1 + EPS)
    tb1 = bpv16 - m1 * sc1
    Wp1f = Wp1_16 * sc1[None, :]
    bp1f = (jnp.zeros((16,), f32).at[0:3].set(bp1)) * sc1 + tb1
    bpf = jnp.zeros((8, C), f32).at[0, 0:16].set(bp1f).at[1].set(bp2)

    s2 = _run_p1(xkg3, xq, prel3, Wp1f, Wp2_16, bpf)
    m2 = s2[0] / cnt
    v2 = s2[1] / cnt - m2 * m2
    sc2 = gw1 / jnp.sqrt(v2 + EPS)
    tb2 = bw1 - m2 * sc2
    st2 = jnp.zeros((8, C), f32).at[0].set(sc2).at[1].set(tb2).at[2, 0:CS].set(bww1)

    u3, s3 = _run_p2(xkg3, xq, prel3, Wp1f, Wp2_16, bpf, st2, Ww1)
    m3 = s3[0] / cnt
    v3 = s3[1] / cnt - m3 * m3
    sc3 = gw2 / jnp.sqrt(v3 + EPS)
    tb3 = bw2 - m3 * sc3
    st3 = jnp.zeros((8, CS), f32).at[0].set(sc3).at[1].set(tb3).at[2].set(bww2)

    out = _run_p3(u3, xvg3, prel3, Wp1f, Wp2_16, bpf, st3, Ww2)
    return out[:NREAL]


# speedup vs baseline: 4.4504x; 1.2995x over previous
import functools

import jax
import jax.numpy as jnp
from jax import lax
from jax.experimental import pallas as pl
from jax.experimental.pallas import tpu as pltpu
from jax.experimental.pallas import tpu_sc as plsc

NREAL = 10000
NPAD = 10240
C = 256
NS = 16
SP = 8
CS = 32
BLK = 128
NBLK = NPAD // BLK
FAR = 1.0e6
BIGI = 1 << 30
EPS = 1e-5


def _qkv_body(x_ref, wq_ref, wk_ref, wv_ref, b3_ref, q_ref, k_ref, v_ref):
    xb = x_ref[...]
    q_ref[...] = jnp.dot(xb, wq_ref[...], preferred_element_type=jnp.float32) + b3_ref[0:1, :]
    k_ref[...] = jnp.dot(xb, wk_ref[...], preferred_element_type=jnp.float32) + b3_ref[1:2, :]
    v_ref[...] = jnp.dot(xb, wv_ref[...], preferred_element_type=jnp.float32) + b3_ref[2:3, :]


def _run_qkv(x_pad, Wq, Wk, Wv, b3):
    out = jax.ShapeDtypeStruct((NPAD, C), jnp.float32)
    return pl.pallas_call(
        _qkv_body,
        grid=(NBLK,),
        in_specs=[
            pl.BlockSpec((BLK, C), lambda i: (i, 0)),
            pl.BlockSpec((C, C), lambda i: (0, 0)),
            pl.BlockSpec((C, C), lambda i: (0, 0)),
            pl.BlockSpec((C, C), lambda i: (0, 0)),
            pl.BlockSpec((8, C), lambda i: (0, 0)),
        ],
        out_specs=[
            pl.BlockSpec((BLK, C), lambda i: (i, 0)),
            pl.BlockSpec((BLK, C), lambda i: (i, 0)),
            pl.BlockSpec((BLK, C), lambda i: (i, 0)),
        ],
        out_shape=[out, out, out],
    )(x_pad, Wq, Wk, Wv, b3)


def _knn_body(pb_ref, pt_ref, idx_ref):
    pb = pb_ref[...]
    pt = pt_ref[...]
    sqb = jnp.sum(pb * pb, axis=1, keepdims=True)
    sqa = jnp.sum(pt * pt, axis=0, keepdims=True)
    d2 = sqb + sqa - 2.0 * jnp.dot(pb, pt, preferred_element_type=jnp.float32)
    li = lax.broadcasted_iota(jnp.int32, (BLK, NPAD), 1)
    cur = d2
    for t in range(NS):
        am = jnp.argmin(cur, axis=1).astype(jnp.int32)[:, None]
        idx_ref[:, t:t + 1] = am
        cur = jnp.where(li == am, jnp.float32(3e38), cur)


def _run_knn(p16, p16t):
    return pl.pallas_call(
        _knn_body,
        grid=(NBLK,),
        in_specs=[
            pl.BlockSpec((BLK, 16), lambda i: (i, 0)),
            pl.BlockSpec((16, NPAD), lambda i: (0, 0)),
        ],
        out_specs=pl.BlockSpec((BLK, NS), lambda i: (i, 0)),
        out_shape=jax.ShapeDtypeStruct((NPAD, NS), jnp.int32),
    )(p16, p16t)


def _sc_gather(xk, xv, p128, idx_flat):
    info = plsc.get_sparse_core_info()
    nw = info.num_cores * info.num_subcores
    B = idx_flat.shape[0]
    b_per_w = B // nw
    CH = 128
    n_ch = b_per_w // CH
    mesh = plsc.VectorSubcoreMesh(core_axis_name="c", subcore_axis_name="s")

    @functools.partial(
        pl.kernel, mesh=mesh,
        out_type=[
            jax.ShapeDtypeStruct((B, C), jnp.float32),
            jax.ShapeDtypeStruct((B, C), jnp.float32),
            jax.ShapeDtypeStruct((B, 128), jnp.float32),
        ],
        scratch_types=[
            pltpu.VMEM((CH,), jnp.int32),
            pltpu.VMEM((CH, C), jnp.float32),
            pltpu.VMEM((CH, C), jnp.float32),
            pltpu.VMEM((CH, 128), jnp.float32),
            pltpu.SemaphoreType.DMA,
            pltpu.SemaphoreType.DMA,
            pltpu.SemaphoreType.DMA,
        ],
    )
    def k(tk_hbm, tv_hbm, tp_hbm, idx_hbm, ok_hbm, ov_hbm, op_hbm,
          idx_v, rk_v, rv_v, rp_v, sk, sv, sp):
        wid = lax.axis_index("s") * info.num_cores + lax.axis_index("c")
        base = wid * b_per_w

        def body(i, carry):
            off = base + i * CH
            pltpu.sync_copy(idx_hbm.at[pl.ds(off, CH)], idx_v)
            ck = pltpu.async_copy(tk_hbm.at[idx_v], rk_v, sk)
            cv = pltpu.async_copy(tv_hbm.at[idx_v], rv_v, sv)
            cp = pltpu.async_copy(tp_hbm.at[idx_v], rp_v, sp)
            ck.wait()
            cv.wait()
            cp.wait()
            pltpu.sync_copy(rk_v, ok_hbm.at[pl.ds(off, CH)])
            pltpu.sync_copy(rv_v, ov_hbm.at[pl.ds(off, CH)])
            pltpu.sync_copy(rp_v, op_hbm.at[pl.ds(off, CH)])
            return carry

        lax.fori_loop(0, n_ch, body, 0)

    return k(xk, xv, p128, idx_flat)


def _pr_block(prel_ref, wp1_ref, wp2_ref, bp_ref):
    prel2 = prel_ref[...].reshape(BLK * NS, 16)
    y = jnp.dot(prel2, wp1_ref[...], preferred_element_type=jnp.float32) + bp_ref[0:1, 0:16]
    y = jnp.maximum(y, 0.0)
    return jnp.dot(y, wp2_ref[...], preferred_element_type=jnp.float32) + bp_ref[1:2, :]


def _rowmask(blk_i):
    pid = lax.broadcasted_iota(jnp.int32, (BLK * NS, 1), 0) // NS + blk_i * BLK
    return jnp.where(pid < NREAL, 1.0, 0.0).astype(jnp.float32)


def _p0_body(prel_ref, wp1_ref, bp1_ref, s_ref):
    @pl.when(pl.program_id(0) == 0)
    def _():
        s_ref[...] = jnp.zeros_like(s_ref)

    prel2 = prel_ref[...].reshape(BLK * NS, 16)
    y = jnp.dot(prel2, wp1_ref[...], preferred_element_type=jnp.float32) + bp1_ref[0:1, :]
    mask = _rowmask(pl.program_id(0))
    ym = y * mask
    s_ref[0:1, :] += jnp.sum(ym, axis=0, keepdims=True)
    s_ref[1:2, :] += jnp.sum(ym * y, axis=0, keepdims=True)


def _run_p0(prel3, Wp1_16, bp1_16):
    return pl.pallas_call(
        _p0_body,
        grid=(NBLK,),
        in_specs=[
            pl.BlockSpec((BLK, NS, 16), lambda i: (i, 0, 0)),
            pl.BlockSpec((16, 16), lambda i: (0, 0)),
            pl.BlockSpec((8, 16), lambda i: (0, 0)),
        ],
        out_specs=pl.BlockSpec((8, 16), lambda i: (0, 0)),
        out_shape=jax.ShapeDtypeStruct((8, 16), jnp.float32),
    )(prel3, Wp1_16, bp1_16)


def _p1_body(xkg_ref, xq_ref, prel_ref, wp1_ref, wp2_ref, bp_ref, s_ref):
    @pl.when(pl.program_id(0) == 0)
    def _():
        s_ref[...] = jnp.zeros_like(s_ref)

    pr = _pr_block(prel_ref, wp1_ref, wp2_ref, bp_ref)
    xkg2 = xkg_ref[...].reshape(BLK * NS, C)
    xq2 = jnp.broadcast_to(xq_ref[...][:, None, :], (BLK, NS, C)).reshape(BLK * NS, C)
    r = xkg2 - xq2 + pr
    mask = _rowmask(pl.program_id(0))
    rm = r * mask
    s_ref[0:1, :] += jnp.sum(rm, axis=0, keepdims=True)
    s_ref[1:2, :] += jnp.sum(rm * r, axis=0, keepdims=True)


def _run_p1(xkg3, xq, prel3, Wp1f, Wp2_16, bpf):
    return pl.pallas_call(
        _p1_body,
        grid=(NBLK,),
        in_specs=[
            pl.BlockSpec((BLK, NS, C), lambda i: (i, 0, 0)),
            pl.BlockSpec((BLK, C), lambda i: (i, 0)),
            pl.BlockSpec((BLK, NS, 16), lambda i: (i, 0, 0)),
            pl.BlockSpec((16, 16), lambda i: (0, 0)),
            pl.BlockSpec((16, C), lambda i: (0, 0)),
            pl.BlockSpec((8, C), lambda i: (0, 0)),
        ],
        out_specs=pl.BlockSpec((8, C), lambda i: (0, 0)),
        out_shape=jax.ShapeDtypeStruct((8, C), jnp.float32),
    )(xkg3, xq, prel3, Wp1f, Wp2_16, bpf)


def _p2_body(xkg_ref, xq_ref, prel_ref, wp1_ref, wp2_ref, bp_ref,
             st2_ref, ww1_ref, u_ref, s_ref):
    @pl.when(pl.program_id(0) == 0)
    def _():
        s_ref[...] = jnp.zeros_like(s_ref)

    pr = _pr_block(prel_ref, wp1_ref, wp2_ref, bp_ref)
    xkg2 = xkg_ref[...].reshape(BLK * NS, C)
    xq2 = jnp.broadcast_to(xq_ref[...][:, None, :], (BLK, NS, C)).reshape(BLK * NS, C)
    r = xkg2 - xq2 + pr
    rn = jnp.maximum(r * st2_ref[0:1, :] + st2_ref[1:2, :], 0.0)
    u = jnp.dot(rn, ww1_ref[...], preferred_element_type=jnp.float32) + st2_ref[2:3, 0:CS]
    u_ref[...] = u.reshape(BLK, NS, CS)
    mask = _rowmask(pl.program_id(0))
    um = u * mask
    s_ref[0:1, :] += jnp.sum(um, axis=0, keepdims=True)
    s_ref[1:2, :] += jnp.sum(um * u, axis=0, keepdims=True)


def _run_p2(xkg3, xq, prel3, Wp1f, Wp2_16, bpf, st2, Ww1):
    return pl.pallas_call(
        _p2_body,
        grid=(NBLK,),
        in_specs=[
            pl.BlockSpec((BLK, NS, C), lambda i: (i, 0, 0)),
            pl.BlockSpec((BLK, C), lambda i: (i, 0)),
            pl.BlockSpec((BLK, NS, 16), lambda i: (i, 0, 0)),
            pl.BlockSpec((16, 16), lambda i: (0, 0)),
            pl.BlockSpec((16, C), lambda i: (0, 0)),
            pl.BlockSpec((8, C), lambda i: (0, 0)),
            pl.BlockSpec((8, C), lambda i: (0, 0)),
            pl.BlockSpec((C, CS), lambda i: (0, 0)),
        ],
        out_specs=[
            pl.BlockSpec((BLK, NS, CS), lambda i: (i, 0, 0)),
            pl.BlockSpec((8, CS), lambda i: (0, 0)),
        ],
        out_shape=[
            jax.ShapeDtypeStruct((NPAD, NS, CS), jnp.float32),
            jax.ShapeDtypeStruct((8, CS), jnp.float32),
        ],
    )(xkg3, xq, prel3, Wp1f, Wp2_16, bpf, st2, Ww1)


def _p3_body(u_ref, xvg_ref, prel_ref, wp1_ref, wp2_ref, bp_ref,
             st3_ref, ww2_ref, o_ref):
    u = u_ref[...].reshape(BLK * NS, CS)
    un = jnp.maximum(u * st3_ref[0:1, 0:CS] + st3_ref[1:2, 0:CS], 0.0)
    w = jnp.dot(un, ww2_ref[...], preferred_element_type=jnp.float32) + st3_ref[2:3, 0:CS]
    w3 = w.reshape(BLK, NS, CS)
    wmax = jnp.max(w3, axis=1, keepdims=True)
    we = jnp.exp(w3 - wmax)
    wsm = we / jnp.sum(we, axis=1, keepdims=True)

    pr = _pr_block(prel_ref, wp1_ref, wp2_ref, bp_ref)
    val = xvg_ref[...] + pr.reshape(BLK, NS, C)
    for s in range(SP):
        seg = val[:, :, s * CS:(s + 1) * CS] * wsm
        o_ref[:, s * CS:(s + 1) * CS] = jnp.sum(seg, axis=1)


def _run_p3(u3, xvg3, prel3, Wp1f, Wp2_16, bpf, st3, Ww2):
    return pl.pallas_call(
        _p3_body,
        grid=(NBLK,),
        in_specs=[
            pl.BlockSpec((BLK, NS, CS), lambda i: (i, 0, 0)),
            pl.BlockSpec((BLK, NS, C), lambda i: (i, 0, 0)),
            pl.BlockSpec((BLK, NS, 16), lambda i: (i, 0, 0)),
            pl.BlockSpec((16, 16), lambda i: (0, 0)),
            pl.BlockSpec((16, C), lambda i: (0, 0)),
            pl.BlockSpec((8, C), lambda i: (0, 0)),
            pl.BlockSpec((8, CS), lambda i: (0, 0)),
            pl.BlockSpec((CS, CS), lambda i: (0, 0)),
        ],
        out_specs=pl.BlockSpec((BLK, C), lambda i: (i, 0)),
        out_shape=jax.ShapeDtypeStruct((NPAD, C), jnp.float32),
    )(u3, xvg3, prel3, Wp1f, Wp2_16, bpf, st3, Ww2)


def kernel(p, x, o, Wq, bq, Wk, bk, Wv, bv, Wp1, bp1, gp, bp, Wp2, bp2,
           gw1, bw1, Ww1, bww1, gw2, bw2, Ww2, bww2):
    f32 = jnp.float32
    npad = NPAD - NREAL

    x_pad = jnp.pad(x, ((0, npad), (0, 0)))
    p16 = jnp.pad(p, ((0, npad), (0, 13)), constant_values=0.0)
    p16 = p16.at[NREAL:, 0:3].set(FAR)
    p16t = p16.T

    b3 = jnp.zeros((8, C), f32).at[0].set(bq).at[1].set(bk).at[2].set(bv)

    Wp1_16 = jnp.zeros((16, 16), f32).at[0:3, 0:3].set(Wp1)
    bp1_16 = jnp.zeros((8, 16), f32).at[0, 0:3].set(bp1)
    Wp2_16 = jnp.zeros((16, C), f32).at[0:3, :].set(Wp2)

    xq, xk, xv = _run_qkv(x_pad, Wq, Wk, Wv, b3)

    idx = _run_knn(p16, p16t)

    p128 = jnp.pad(p16, ((0, 0), (0, 112)))
    xkg, xvg, pg = _sc_gather(xk, xv, p128, idx.reshape(-1))
    xkg3 = xkg.reshape(NPAD, NS, C)
    xvg3 = xvg.reshape(NPAD, NS, C)
    prel3 = pg.reshape(NPAD, NS, 128)[:, :, 0:16] - p16[:, None, :]

    s1 = _run_p0(prel3, Wp1_16, bp1_16)
    cnt = jnp.float32(NREAL * NS)
    m1 = s1[0] / cnt
    v1 = s1[1] / cnt - m1 * m1
    gp16 = jnp.ones((16,), f32).at[0:3].set(gp)
    bpv16 = jnp.zeros((16,), f32).at[0:3].set(bp)
    sc1 = gp16 / jnp.sqrt(v1 + EPS)
    tb1 = bpv16 - m1 * sc1
    Wp1f = Wp1_16 * sc1[None, :]
    bp1f = (jnp.zeros((16,), f32).at[0:3].set(bp1)) * sc1 + tb1
    bpf = jnp.zeros((8, C), f32).at[0, 0:16].set(bp1f).at[1].set(bp2)

    s2 = _run_p1(xkg3, xq, prel3, Wp1f, Wp2_16, bpf)
    m2 = s2[0] / cnt
    v2 = s2[1] / cnt - m2 * m2
    sc2 = gw1 / jnp.sqrt(v2 + EPS)
    tb2 = bw1 - m2 * sc2
    st2 = jnp.zeros((8, C), f32).at[0].set(sc2).at[1].set(tb2).at[2, 0:CS].set(bww1)

    u3, s3 = _run_p2(xkg3, xq, prel3, Wp1f, Wp2_16, bpf, st2, Ww1)
    m3 = s3[0] / cnt
    v3 = s3[1] / cnt - m3 * m3
    sc3 = gw2 / jnp.sqrt(v3 + EPS)
    tb3 = bw2 - m3 * sc3
    st3 = jnp.zeros((8, CS), f32).at[0].set(sc3).at[1].set(tb3).at[2].set(bww2)

    out = _run_p3(u3, xvg3, prel3, Wp1f, Wp2_16, bpf, st3, Ww2)
    return out[:NREAL]
